# baseline (speedup 1.0000x reference)
import functools

import jax
import jax.numpy as jnp
from jax import lax
from jax.experimental import pallas as pl
from jax.experimental.pallas import tpu as pltpu
from jax.experimental.pallas import tpu_sc as plsc

N = 20000
CONF = 0.5
NC = 2
NS = 16
L = 16
NW = NC * NS
CHUNK = 640
LAST = N - (NW - 1) * CHUNK
NITER = CHUNK // L

_mesh = plsc.VectorSubcoreMesh(
    core_axis_name="c", subcore_axis_name="s", num_cores=NC, num_subcores=NS
)


@functools.partial(
    pl.kernel,
    out_type=jax.ShapeDtypeStruct((NW, L), jnp.float32),
    mesh=_mesh,
    compiler_params=pltpu.CompilerParams(
        needs_layout_passes=False, skip_device_barrier=True
    ),
    scratch_types=[
        pltpu.VMEM((CHUNK,), jnp.float32),
        pltpu.VMEM((CHUNK // L, L, 2), jnp.float32),
        pltpu.VMEM((L,), jnp.float32),
    ],
)
def _masked_sum_sc(cls_hbm, ep_hbm, out_hbm, cls_v, ep_v, acc_v):
    wid = lax.axis_index("s") * NC + lax.axis_index("c")
    base = wid * CHUNK
    ep3_hbm = ep_hbm.reshape(N // L, L, 2)
    grp = wid * (CHUNK // L)

    @pl.when(wid < NW - 1)
    def _():
        pltpu.sync_copy(cls_hbm.at[pl.ds(base, CHUNK)], cls_v)
        pltpu.sync_copy(ep3_hbm.at[pl.ds(grp, CHUNK // L)], ep_v)

    @pl.when(wid == NW - 1)
    def _():
        pltpu.sync_copy(cls_hbm.at[pl.ds(base, LAST)], cls_v.at[pl.ds(0, LAST)])
        pltpu.sync_copy(
            ep3_hbm.at[pl.ds(grp, LAST // L)], ep_v.at[pl.ds(0, LAST // L)]
        )

    lanes = lax.iota(jnp.int32, L)
    zeros = jnp.zeros((L,), jnp.int32)

    def body(k, acc):
        off = k * L
        c = cls_v[pl.ds(off, L)]
        kvec = zeros + k
        e0 = plsc.load_gather(ep_v, [kvec, lanes, zeros])
        e1 = plsc.load_gather(ep_v, [kvec, lanes, zeros + 1])
        valid = (base + off + lanes < N) & (c >= CONF)
        return acc + jnp.where(valid, c + e0 + e1, 0.0)

    acc = lax.fori_loop(0, NITER, body, jnp.zeros((L,), jnp.float32))
    acc_v[...] = acc
    pltpu.sync_copy(acc_v, out_hbm.at[wid])


def kernel(cls_scores, end_points):
    partials = _masked_sum_sc(cls_scores, end_points)
    return jnp.sum(partials)


# speedup vs baseline: 1.3130x; 1.3130x over previous
import functools

import jax
import jax.numpy as jnp
from jax import lax
from jax.experimental import pallas as pl
from jax.experimental.pallas import tpu as pltpu
from jax.experimental.pallas import tpu_sc as plsc

N = 20000
CONF = 0.5
NC = 2
NS = 16
L = 16
NW = NC * NS
CHUNK = 640
LAST = N - (NW - 1) * CHUNK
NITER = CHUNK // L

_mesh = plsc.VectorSubcoreMesh(
    core_axis_name="c", subcore_axis_name="s", num_cores=NC, num_subcores=NS
)


@functools.partial(
    pl.kernel,
    out_type=jax.ShapeDtypeStruct((NW, L), jnp.float32),
    mesh=_mesh,
    compiler_params=pltpu.CompilerParams(
        needs_layout_passes=False,
        skip_device_barrier=True,
    ),
    scratch_types=[
        pltpu.VMEM((CHUNK,), jnp.float32),
        pltpu.VMEM((CHUNK,), jnp.float32),
        pltpu.VMEM((CHUNK,), jnp.float32),
        pltpu.VMEM((L,), jnp.float32),
    ],
)
def _masked_sum_sc(cls_hbm, x_hbm, y_hbm, out_hbm, cls_v, x_v, y_v, acc_v):
    wid = lax.axis_index("s") * NC + lax.axis_index("c")
    base = wid * CHUNK

    @pl.when(wid < NW - 1)
    def _():
        pltpu.sync_copy(cls_hbm.at[pl.ds(base, CHUNK)], cls_v)
        pltpu.sync_copy(x_hbm.at[pl.ds(base, CHUNK)], x_v)
        pltpu.sync_copy(y_hbm.at[pl.ds(base, CHUNK)], y_v)

    @pl.when(wid == NW - 1)
    def _():
        pltpu.sync_copy(cls_hbm.at[pl.ds(base, LAST)], cls_v.at[pl.ds(0, LAST)])
        pltpu.sync_copy(x_hbm.at[pl.ds(base, LAST)], x_v.at[pl.ds(0, LAST)])
        pltpu.sync_copy(y_hbm.at[pl.ds(base, LAST)], y_v.at[pl.ds(0, LAST)])

    lanes = lax.iota(jnp.int32, L)

    def body(k, acc):
        off = k * L
        c = cls_v[pl.ds(off, L)]
        e0 = x_v[pl.ds(off, L)]
        e1 = y_v[pl.ds(off, L)]
        valid = (base + off + lanes < N) & (c >= CONF)
        return acc + jnp.where(valid, c + e0 + e1, 0.0)

    acc = lax.fori_loop(0, NITER, body, jnp.zeros((L,), jnp.float32))
    acc_v[...] = acc
    pltpu.sync_copy(acc_v, out_hbm.at[wid])


def kernel(cls_scores, end_points):
    partials = _masked_sum_sc(cls_scores, end_points[:, 0], end_points[:, 1])
    return jnp.sum(partials)
